# baseline (speedup 1.0000x reference)
import functools

import jax
import jax.numpy as jnp
from jax import lax
from jax.experimental import pallas as pl
from jax.experimental.pallas import tpu as pltpu
from jax.experimental.pallas import tpu_sc as plsc

N = 1048576
_INFO = plsc.get_sparse_core_info()
NC = _INFO.num_cores
NS = _INFO.num_subcores
NW = NC * NS
CH = N // NW
NB = 1024
NV = CH // 16
KMAX = (1 << 30) - 1
LN2 = 0.6931471805599453

_MESH = plsc.VectorSubcoreMesh(core_axis_name="c", subcore_axis_name="s")
_CPARAMS = pltpu.CompilerParams(needs_layout_passes=False)

_GATHER_DNUMS = lax.GatherDimensionNumbers(
    offset_dims=(), collapsed_slice_dims=(0,), start_index_map=(0,)
)


def _lane_bcast_last(x):
    idx = jnp.full((16, 1), 15, jnp.int32)
    return lax.gather(
        x, idx, _GATHER_DNUMS, slice_sizes=(1,),
        mode=lax.GatherScatterMode.PROMISE_IN_BOUNDS,
    )


def _worker_id():
    return lax.axis_index("s") * NC + lax.axis_index("c")


def _zero_i32(ref, n):
    z = jnp.zeros((16,), jnp.int32)

    def body(k, c):
        ref[pl.ds(k * 16, 16)] = z
        return c

    lax.fori_loop(0, n // 16, body, 0)


def _merge_subhist(sub_v, m_v):
    def body(c, carry):
        acc = jnp.zeros((16,), jnp.int32)
        for lane in range(16):
            acc = acc + sub_v[pl.ds(lane * NB + c * 16, 16)]
        m_v[pl.ds(c * 16, 16)] = acc
        return carry

    lax.fori_loop(0, NB // 16, body, 0)


@functools.partial(
    pl.kernel,
    mesh=_MESH,
    compiler_params=_CPARAMS,
    out_type=(
        jax.ShapeDtypeStruct((N,), jnp.int32),
        jax.ShapeDtypeStruct((N,), jnp.float32),
        jax.ShapeDtypeStruct((NW, NB), jnp.int32),
        jax.ShapeDtypeStruct((NW, 4, 16), jnp.float32),
    ),
    scratch_types=[
        pltpu.VMEM((CH,), jnp.float32),
        pltpu.VMEM((CH,), jnp.float32),
        pltpu.VMEM((CH,), jnp.int32),
        pltpu.VMEM((16 * NB,), jnp.int32),
        pltpu.VMEM((NB,), jnp.int32),
        pltpu.VMEM((4, 16), jnp.float32),
        pltpu.SemaphoreType.DMA,
    ],
)
def _k_pre(t_hbm, r_hbm, e_hbm, key_hbm, sw_hbm, hist_hbm, part_hbm,
           t_v, r_v, e_v, sub_v, m_v, part_v, sem):
    wid = _worker_id()
    base = wid * CH
    pltpu.sync_copy(t_hbm.at[pl.ds(base, CH)], t_v)
    pltpu.sync_copy(r_hbm.at[pl.ds(base, CH)], r_v)
    pltpu.sync_copy(e_hbm.at[pl.ds(base, CH)], e_v)
    _zero_i32(sub_v, 16 * NB)
    lanes = lax.iota(jnp.int32, 16)
    ones = jnp.ones((16,), jnp.int32)

    def body(k, accs):
        ae, aer = accs
        sl = pl.ds(k * 16, 16)
        e16 = e_v[sl]
        r16 = r_v[sl]
        t16 = t_v[sl]
        ef = e16.astype(jnp.float32)
        ww = jnp.exp(r16)
        r_v[sl] = ww * (1.0 - 2.0 * ef)
        key16 = KMAX - plsc.bitcast(t16, jnp.int32)
        e_v[sl] = key16
        d16 = key16 & (NB - 1)
        plsc.addupdate_scatter(sub_v, [lanes * NB + d16], ones)
        return (ae + ef, aer + ef * r16)

    z = jnp.zeros((16,), jnp.float32)
    ae, aer = lax.fori_loop(0, NV, body, (z, z))
    _merge_subhist(sub_v, m_v)
    part_v[0, :] = z
    part_v[1, :] = ae
    part_v[2, :] = aer
    part_v[3, :] = z
    pltpu.sync_copy(r_v, sw_hbm.at[pl.ds(base, CH)])
    pltpu.sync_copy(e_v, key_hbm.at[pl.ds(base, CH)])
    pltpu.sync_copy(m_v, hist_hbm.at[wid])
    pltpu.sync_copy(part_v, part_hbm.at[wid])


def _make_hist(shift):
    @functools.partial(
        pl.kernel,
        mesh=_MESH,
        compiler_params=_CPARAMS,
        out_type=jax.ShapeDtypeStruct((NW, NB), jnp.int32),
        scratch_types=[
            pltpu.VMEM((CH,), jnp.int32),
            pltpu.VMEM((16 * NB,), jnp.int32),
            pltpu.VMEM((NB,), jnp.int32),
            pltpu.SemaphoreType.DMA,
        ],
    )
    def _k_hist(key_hbm, hist_hbm, k_v, sub_v, m_v, sem):
        wid = _worker_id()
        base = wid * CH
        pltpu.sync_copy(key_hbm.at[pl.ds(base, CH)], k_v)
        _zero_i32(sub_v, 16 * NB)
        lanes = lax.iota(jnp.int32, 16)
        ones = jnp.ones((16,), jnp.int32)

        def body(k, c):
            d16 = (k_v[pl.ds(k * 16, 16)] >> shift) & (NB - 1)
            plsc.addupdate_scatter(sub_v, [lanes * NB + d16], ones)
            return c

        lax.fori_loop(0, NV, body, 0)
        _merge_subhist(sub_v, m_v)
        pltpu.sync_copy(m_v, hist_hbm.at[wid])

    return _k_hist


def _make_scat(shift, write_keys):
    outs = [jax.ShapeDtypeStruct((N,), jnp.float32)]
    if write_keys:
        outs.append(jax.ShapeDtypeStruct((N,), jnp.int32))

    @functools.partial(
        pl.kernel,
        mesh=_MESH,
        compiler_params=_CPARAMS,
        out_type=tuple(outs) if write_keys else outs[0],
        scratch_types=[
            pltpu.VMEM((CH,), jnp.int32),
            pltpu.VMEM((CH,), jnp.float32),
            pltpu.VMEM((CH // 128, 128), jnp.int32),
            pltpu.VMEM((NB,), jnp.int32),
            pltpu.VMEM((NB,), jnp.int32),
            pltpu.VMEM((NB,), jnp.int32),
            pltpu.VMEM((NB,), jnp.int32),
            pltpu.SemaphoreType.DMA,
        ],
    )
    def _k_scat(key_hbm, sw_hbm, hist_hbm, *rest):
        if write_keys:
            swo_hbm, keyo_hbm = rest[0], rest[1]
            scratch = rest[2:]
        else:
            swo_hbm = rest[0]
            keyo_hbm = None
            scratch = rest[1:]
        k_v, s_v, p_v, offs_v, tot_v, less_v, hrow_v, sem = scratch
        wid = _worker_id()
        base = wid * CH
        pltpu.sync_copy(key_hbm.at[pl.ds(base, CH)], k_v)
        pltpu.sync_copy(sw_hbm.at[pl.ds(base, CH)], s_v)
        _zero_i32(tot_v, NB)
        _zero_i32(less_v, NB)
        wid_vec = jnp.full((16,), wid, jnp.int32)
        zi = jnp.zeros((16,), jnp.int32)
        for w in range(NW):
            pltpu.sync_copy(hist_hbm.at[w], hrow_v)
            selv = jnp.full((16,), w, jnp.int32) < wid_vec

            def acc_body(c, carry, _selv=selv):
                sl = pl.ds(c * 16, 16)
                h16 = hrow_v[sl]
                tot_v[sl] = tot_v[sl] + h16
                less_v[sl] = less_v[sl] + jnp.where(_selv, h16, zi)
                return carry

            lax.fori_loop(0, NB // 16, acc_body, 0)

        def pfx_body(c, carry):
            sl = pl.ds(c * 16, 16)
            t16 = tot_v[sl]
            pre = jnp.cumsum(t16) + carry
            offs_v[sl] = (pre - t16) + less_v[sl]
            return _lane_bcast_last(pre)

        lax.fori_loop(0, NB // 16, pfx_body, zi)

        def rank_body(k, c):
            k16 = k_v[pl.ds(k * 16, 16)]
            d16 = (k16 >> shift) & (NB - 1)
            sc, mlast = plsc.scan_count(d16)
            pos = plsc.load_gather(offs_v, [d16]) + sc - 1
            plsc.addupdate_scatter(offs_v, [d16], sc, mask=mlast)
            p_v[k // 8, pl.ds((k % 8) * 16, 16)] = pos
            return c

        lax.fori_loop(0, NV, rank_body, 0)

        n_arr = 2 if write_keys else 1
        LAG = 4

        def _drain_one():
            pltpu.make_async_copy(
                swo_hbm.at[pl.ds(0, 128)], s_v.at[pl.ds(0, 128)], sem
            ).wait()

        def dma_body(j, c):
            for u in range(4):
                row = j * 4 + u
                pltpu.async_copy(
                    s_v.at[pl.ds(row * 128, 128)], swo_hbm.at[p_v.at[row]], sem)
                if write_keys:
                    pltpu.async_copy(
                        k_v.at[pl.ds(row * 128, 128)], keyo_hbm.at[p_v.at[row]],
                        sem)

            @pl.when(j >= LAG)
            def _():
                for _ in range(4 * n_arr):
                    _drain_one()

            return c

        lax.fori_loop(0, (CH // 128) // 4, dma_body, 0)
        for _ in range(LAG * 4 * n_arr):
            _drain_one()

    return _k_scat


@functools.partial(
    pl.kernel,
    mesh=_MESH,
    compiler_params=_CPARAMS,
    out_type=jax.ShapeDtypeStruct((NW, 4, 16), jnp.float32),
    scratch_types=[
        pltpu.VMEM((CH,), jnp.float32),
        pltpu.VMEM((4, 16), jnp.float32),
        pltpu.SemaphoreType.DMA,
    ],
)
def _k_sums(sw_hbm, part_hbm, s_v, part_v, sem):
    wid = _worker_id()
    pltpu.sync_copy(sw_hbm.at[pl.ds(wid * CH, CH)], s_v)

    def body(k, acc):
        b = plsc.bitcast(s_v[pl.ds(k * 16, 16)], jnp.int32)
        return acc + plsc.bitcast(b & 0x7FFFFFFF, jnp.float32)

    z = jnp.zeros((16,), jnp.float32)
    acc = lax.fori_loop(0, NV, body, z)
    part_v[0, :] = acc
    part_v[1, :] = z
    part_v[2, :] = z
    part_v[3, :] = z
    pltpu.sync_copy(part_v, part_hbm.at[wid])


@functools.partial(
    pl.kernel,
    mesh=_MESH,
    compiler_params=_CPARAMS,
    out_type=jax.ShapeDtypeStruct((NW, 16), jnp.float32),
    scratch_types=[
        pltpu.VMEM((CH,), jnp.float32),
        pltpu.VMEM((NW, 4, 16), jnp.float32),
        pltpu.VMEM((16,), jnp.float32),
        pltpu.SemaphoreType.DMA,
    ],
)
def _k_log(w_hbm, part_hbm, out_hbm, w_v, part_v, out_v, sem):
    wid = _worker_id()
    base = wid * CH
    pltpu.sync_copy(w_hbm.at[pl.ds(base, CH)], w_v)
    pltpu.sync_copy(part_hbm, part_v)

    wid_vec = jnp.full((16,), wid, jnp.int32)
    pacc = jnp.zeros((16,), jnp.float32)
    for v in range(NW):
        sel = jnp.full((16,), v, jnp.int32) < wid_vec
        pacc = pacc + jnp.where(sel, part_v[v, 0, :], 0.0)
    carry0 = _lane_bcast_last(jnp.cumsum(pacc))

    def body(k, st):
        cvec, acc = st
        swv = w_v[pl.ds(k * 16, 16)]
        b = plsc.bitcast(swv, jnp.int32)
        ww = plsc.bitcast(b & 0x7FFFFFFF, jnp.float32)
        ef = lax.shift_right_logical(b, 31).astype(jnp.float32)
        pre = jnp.cumsum(ww) + cvec
        cnew = _lane_bcast_last(pre)
        pb = plsc.bitcast(pre, jnp.int32)
        ex = lax.shift_right_logical(pb, 23) - 127
        m = plsc.bitcast((pb & 0x7FFFFF) | 0x3F800000, jnp.float32)
        big = m >= 1.5
        m = jnp.where(big, m * 0.5, m)
        exf = (ex + big.astype(jnp.int32)).astype(jnp.float32)
        s = (m - 1.0) / (m + 1.0)
        s2 = s * s
        lnm = 2.0 * s * (1.0 + s2 * (1.0 / 3.0 + s2 * 0.2))
        lnx = exf * LN2 + lnm
        return (cnew, acc + ef * lnx)

    _, acc = lax.fori_loop(
        0, NV, body, (carry0, jnp.zeros((16,), jnp.float32))
    )
    out_v[...] = acc
    pltpu.sync_copy(out_v, out_hbm.at[wid])


_scat1 = _make_scat(0, True)
_hist2 = _make_hist(10)
_scat2 = _make_scat(10, True)
_hist3 = _make_hist(20)
_scat3 = _make_scat(20, False)


def kernel(risk_scores, events, survival_times):
    key0, sw0, hist1, parts = _k_pre(survival_times, risk_scores, events)
    sw1, key1 = _scat1(key0, sw0, hist1)
    hist2 = _hist2(key1)
    sw2, key2 = _scat2(key1, sw1, hist2)
    hist3 = _hist3(key2)
    sw3 = _scat3(key2, sw2, hist3)
    parts2 = _k_sums(sw3)
    accs = _k_log(sw3, parts2)
    sum_e = parts[:, 1, :].sum()
    sum_er = parts[:, 2, :].sum()
    sum_elogc = accs.sum()
    return (sum_elogc - sum_er) / sum_e


# speedup vs baseline: 3.0655x; 3.0655x over previous
import functools

import jax
import jax.numpy as jnp
from jax import lax
from jax.experimental import pallas as pl
from jax.experimental.pallas import tpu as pltpu
from jax.experimental.pallas import tpu_sc as plsc

N = 1048576
_INFO = plsc.get_sparse_core_info()
NC = _INFO.num_cores
NS = _INFO.num_subcores
NW = NC * NS
CH = N // NW
NB = 1024
NR = NW * NB
NV = CH // 16
KMAX = (1 << 30) - 1
LN2 = 0.6931471805599453

_MESH = plsc.VectorSubcoreMesh(core_axis_name="c", subcore_axis_name="s")
_CPARAMS = pltpu.CompilerParams(needs_layout_passes=False)

_GATHER_DNUMS = lax.GatherDimensionNumbers(
    offset_dims=(), collapsed_slice_dims=(0,), start_index_map=(0,)
)


def _lane_bcast_last(x):
    idx = jnp.full((16, 1), 15, jnp.int32)
    return lax.gather(
        x, idx, _GATHER_DNUMS, slice_sizes=(1,),
        mode=lax.GatherScatterMode.PROMISE_IN_BOUNDS,
    )


def _worker_id():
    return lax.axis_index("s") * NC + lax.axis_index("c")


def _zero_i32(ref, n):
    z = jnp.zeros((16,), jnp.int32)

    def body(k, c):
        ref[pl.ds(k * 16, 16)] = z
        return c

    lax.fori_loop(0, n // 16, body, 0)


@functools.partial(
    pl.kernel,
    mesh=_MESH,
    compiler_params=_CPARAMS,
    out_type=(
        jax.ShapeDtypeStruct((N,), jnp.float32),
        jax.ShapeDtypeStruct((N,), jnp.float32),
        jax.ShapeDtypeStruct((NW, 4, 16), jnp.float32),
    ),
    scratch_types=[
        pltpu.VMEM((CH,), jnp.float32),
        pltpu.VMEM((CH,), jnp.float32),
        pltpu.VMEM((CH,), jnp.int32),
        pltpu.VMEM((4, 16), jnp.float32),
        pltpu.SemaphoreType.DMA,
    ],
)
def _k_pre(t_hbm, r_hbm, e_hbm, key_hbm, sw_hbm, part_hbm,
           t_v, r_v, e_v, part_v, sem):
    wid = _worker_id()
    base = wid * CH
    pltpu.sync_copy(t_hbm.at[pl.ds(base, CH)], t_v)
    pltpu.sync_copy(r_hbm.at[pl.ds(base, CH)], r_v)
    pltpu.sync_copy(e_hbm.at[pl.ds(base, CH)], e_v)

    def body(k, accs):
        ae, aer = accs
        sl = pl.ds(k * 16, 16)
        e16 = e_v[sl]
        r16 = r_v[sl]
        t16 = t_v[sl]
        ef = e16.astype(jnp.float32)
        ww = jnp.exp(r16)
        r_v[sl] = ww * (1.0 - 2.0 * ef)
        t_v[sl] = plsc.bitcast(KMAX - plsc.bitcast(t16, jnp.int32),
                               jnp.float32)
        return (ae + ef, aer + ef * r16)

    z = jnp.zeros((16,), jnp.float32)
    ae, aer = lax.fori_loop(0, NV, body, (z, z))
    part_v[0, :] = z
    part_v[1, :] = ae
    part_v[2, :] = aer
    part_v[3, :] = z
    pltpu.sync_copy(r_v, sw_hbm.at[pl.ds(base, CH)])
    pltpu.sync_copy(t_v, key_hbm.at[pl.ds(base, CH)])
    pltpu.sync_copy(part_v, part_hbm.at[wid])


def _make_group(shift):
    @functools.partial(
        pl.kernel,
        mesh=_MESH,
        compiler_params=_CPARAMS,
        out_type=(
            jax.ShapeDtypeStruct((N,), jnp.float32),
            jax.ShapeDtypeStruct((N,), jnp.float32),
            jax.ShapeDtypeStruct((NW, NB), jnp.int32),
        ),
        scratch_types=[
            pltpu.VMEM((CH,), jnp.float32),
            pltpu.VMEM((CH,), jnp.float32),
            pltpu.VMEM((CH,), jnp.float32),
            pltpu.VMEM((16 * NB,), jnp.int32),
            pltpu.VMEM((NB,), jnp.int32),
            pltpu.VMEM((NB,), jnp.int32),
            pltpu.SemaphoreType.DMA,
        ],
    )
    def _k_group(key_hbm, sw_hbm, keyo_hbm, swo_hbm, hist_hbm,
                 k_v, s_v, g_v, sub_v, m_v, offs_v, sem):
        wid = _worker_id()
        base = wid * CH
        pltpu.sync_copy(key_hbm.at[pl.ds(base, CH)], k_v)
        pltpu.sync_copy(sw_hbm.at[pl.ds(base, CH)], s_v)
        _zero_i32(sub_v, 16 * NB)
        lanes = lax.iota(jnp.int32, 16)
        ones = jnp.ones((16,), jnp.int32)

        def hist_body(k, c):
            kb = plsc.bitcast(k_v[pl.ds(k * 16, 16)], jnp.int32)
            d16 = (kb >> shift) & (NB - 1)
            plsc.addupdate_scatter(sub_v, [lanes * NB + d16], ones)
            return c

        lax.fori_loop(0, NV, hist_body, 0)

        def merge_body(c, carry):
            acc = jnp.zeros((16,), jnp.int32)
            for lane in range(16):
                acc = acc + sub_v[pl.ds(lane * NB + c * 16, 16)]
            m_v[pl.ds(c * 16, 16)] = acc
            return carry

        lax.fori_loop(0, NB // 16, merge_body, 0)

        def prefix_into_offs():
            def pfx(c, carry):
                sl = pl.ds(c * 16, 16)
                t16 = m_v[sl]
                pre = jnp.cumsum(t16) + carry
                offs_v[sl] = pre - t16
                return _lane_bcast_last(pre)

            lax.fori_loop(0, NB // 16, pfx, jnp.zeros((16,), jnp.int32))

        def scat_round(src_ref):
            def body(k, c):
                sl = pl.ds(k * 16, 16)
                kb = plsc.bitcast(k_v[sl], jnp.int32)
                d16 = (kb >> shift) & (NB - 1)
                sc, mlast = plsc.scan_count(d16)
                pos = plsc.load_gather(offs_v, [d16]) + sc - 1
                plsc.addupdate_scatter(offs_v, [d16], sc, mask=mlast)
                plsc.store_scatter(g_v, [pos], src_ref[sl])
                return c

            lax.fori_loop(0, NV, body, 0)

        prefix_into_offs()
        scat_round(k_v)
        pltpu.sync_copy(g_v, keyo_hbm.at[pl.ds(base, CH)])
        prefix_into_offs()
        scat_round(s_v)
        pltpu.sync_copy(g_v, swo_hbm.at[pl.ds(base, CH)])
        pltpu.sync_copy(m_v, hist_hbm.at[wid])

    return _k_group


def _make_build(shift, last):
    outs = [jax.ShapeDtypeStruct((N,), jnp.float32)]
    if last:
        outs.append(jax.ShapeDtypeStruct((NW, 4, 16), jnp.float32))
    else:
        outs.append(jax.ShapeDtypeStruct((N,), jnp.float32))

    @functools.partial(
        pl.kernel,
        mesh=_MESH,
        compiler_params=_CPARAMS,
        out_type=tuple(outs),
        scratch_types=[
            pltpu.VMEM((NR,), jnp.int32),
            pltpu.VMEM((NR,), jnp.float32),
            pltpu.VMEM((CH,), jnp.int32),
            pltpu.VMEM((4, 16), jnp.float32),
            pltpu.SemaphoreType.DMA,
        ],
    )
    def _k_build(keyg_hbm, swg_hbm, hist_hbm, swo_hbm, aux_hbm,
                 lp_v, vd_v, ri_v, part_v, sem):
        wid = _worker_id()
        base = wid * CH
        for w in range(NW):
            pltpu.sync_copy(hist_hbm.at[w], lp_v.at[pl.ds(w * NB, NB)])

        for w in range(NW):
            def pfx(c, carry, _w=w):
                sl = pl.ds(_w * NB + c * 16, 16)
                t16 = lp_v[sl]
                pre = jnp.cumsum(t16) + carry
                lp_v[sl] = pre - t16
                return _lane_bcast_last(pre)

            lax.fori_loop(0, NB // 16, pfx, jnp.zeros((16,), jnp.int32))

        _zero_i32(ri_v, CH)
        lanes = lax.iota(jnp.int32, 16)
        zi = jnp.zeros((16,), jnp.int32)
        base_vec = jnp.full((16,), base, jnp.int32)

        def run_body(q, st):
            gcarry, cover = st
            rho = q * 16 + lanes
            d16 = rho >> 5
            w16 = rho & (NW - 1)
            a16 = w16 * NB + d16
            lpv = plsc.load_gather(lp_v, [a16])
            is_last_d = d16 == (NB - 1)
            a2 = jnp.where(is_last_d, a16, a16 + 1)
            nxt = plsc.load_gather(lp_v, [a2])
            len16 = jnp.where(is_last_d, CH - lpv, nxt - lpv)
            pre = jnp.cumsum(len16) + gcarry
            g16 = pre - len16
            vd_v[pl.ds(q * 16, 16)] = plsc.bitcast(
                w16 * CH + lpv - g16, jnp.float32)
            real = len16 > zi
            inb = real & (g16 >= base_vec) & (g16 < base_vec + CH)
            plsc.store_scatter(ri_v, [g16 - base_vec], rho + 1, mask=inb)
            covc = real & (g16 <= base_vec)
            cover = jnp.maximum(cover, jnp.where(covc, rho + 1, zi))
            return (_lane_bcast_last(pre), cover)

        _, cover = lax.fori_loop(0, NR // 16, run_body, (zi, zi))
        fcarry0 = _lane_bcast_last(plsc.cummax(cover))

        def fill_body(k, fcarry):
            sl = pl.ds(k * 16, 16)
            filled = jnp.maximum(plsc.cummax(ri_v[sl]), fcarry)
            v16 = plsc.bitcast(
                plsc.load_gather(vd_v, [filled - 1]), jnp.int32)
            ri_v[sl] = v16 + (base + k * 16 + lanes)
            return _lane_bcast_last(filled)

        lax.fori_loop(0, NV, fill_body, fcarry0)

        def gather_to(src_hbm, dst_v):
            def body(j, c):
                sl = pl.ds(j * 128, 128)
                pltpu.async_copy(
                    src_hbm.at[ri_v.at[sl]], dst_v.at[sl], sem).wait()
                return c

            lax.fori_loop(0, CH // 128, body, 0)

        gather_to(swg_hbm, vd_v)
        pltpu.sync_copy(vd_v, swo_hbm.at[pl.ds(base, CH)])
        if last:
            def sum_body(k, acc):
                b = plsc.bitcast(vd_v[pl.ds(k * 16, 16)], jnp.int32)
                return acc + plsc.bitcast(b & 0x7FFFFFFF, jnp.float32)

            zf = jnp.zeros((16,), jnp.float32)
            acc = lax.fori_loop(0, NV, sum_body, zf)
            part_v[0, :] = acc
            part_v[1, :] = zf
            part_v[2, :] = zf
            part_v[3, :] = zf
            pltpu.sync_copy(part_v, aux_hbm.at[wid])
        else:
            gather_to(keyg_hbm, vd_v)
            pltpu.sync_copy(vd_v, aux_hbm.at[pl.ds(base, CH)])

    return _k_build


@functools.partial(
    pl.kernel,
    mesh=_MESH,
    compiler_params=_CPARAMS,
    out_type=jax.ShapeDtypeStruct((NW, 16), jnp.float32),
    scratch_types=[
        pltpu.VMEM((CH,), jnp.float32),
        pltpu.VMEM((NW, 4, 16), jnp.float32),
        pltpu.VMEM((16,), jnp.float32),
        pltpu.SemaphoreType.DMA,
    ],
)
def _k_log(w_hbm, part_hbm, out_hbm, w_v, part_v, out_v, sem):
    wid = _worker_id()
    base = wid * CH
    pltpu.sync_copy(w_hbm.at[pl.ds(base, CH)], w_v)
    pltpu.sync_copy(part_hbm, part_v)

    wid_vec = jnp.full((16,), wid, jnp.int32)
    pacc = jnp.zeros((16,), jnp.float32)
    for v in range(NW):
        sel = jnp.full((16,), v, jnp.int32) < wid_vec
        pacc = pacc + jnp.where(sel, part_v[v, 0, :], 0.0)
    carry0 = _lane_bcast_last(jnp.cumsum(pacc))

    def body(k, st):
        cvec, acc = st
        swv = w_v[pl.ds(k * 16, 16)]
        b = plsc.bitcast(swv, jnp.int32)
        ww = plsc.bitcast(b & 0x7FFFFFFF, jnp.float32)
        ef = lax.shift_right_logical(b, 31).astype(jnp.float32)
        pre = jnp.cumsum(ww) + cvec
        cnew = _lane_bcast_last(pre)
        pb = plsc.bitcast(pre, jnp.int32)
        ex = lax.shift_right_logical(pb, 23) - 127
        m = plsc.bitcast((pb & 0x7FFFFF) | 0x3F800000, jnp.float32)
        big = m >= 1.5
        m = jnp.where(big, m * 0.5, m)
        exf = (ex + big.astype(jnp.int32)).astype(jnp.float32)
        s = (m - 1.0) / (m + 1.0)
        s2 = s * s
        lnm = 2.0 * s * (1.0 + s2 * (1.0 / 3.0 + s2 * 0.2))
        lnx = exf * LN2 + lnm
        return (cnew, acc + ef * lnx)

    _, acc = lax.fori_loop(
        0, NV, body, (carry0, jnp.zeros((16,), jnp.float32))
    )
    out_v[...] = acc
    pltpu.sync_copy(out_v, out_hbm.at[wid])


_group1 = _make_group(0)
_build1 = _make_build(0, False)
_group2 = _make_group(10)
_build2 = _make_build(10, False)
_group3 = _make_group(20)
_build3 = _make_build(20, True)


def kernel(risk_scores, events, survival_times):
    key0, sw0, parts = _k_pre(survival_times, risk_scores, events)
    kg1, sg1, h1 = _group1(key0, sw0)
    sw1, key1 = _build1(kg1, sg1, h1)
    kg2, sg2, h2 = _group2(key1, sw1)
    sw2, key2 = _build2(kg2, sg2, h2)
    kg3, sg3, h3 = _group3(key2, sw2)
    sw3, parts2 = _build3(kg3, sg3, h3)
    accs = _k_log(sw3, parts2)
    sum_e = parts[:, 1, :].sum()
    sum_er = parts[:, 2, :].sum()
    sum_elogc = accs.sum()
    return (sum_elogc - sum_er) / sum_e


# speedup vs baseline: 4.7558x; 1.5514x over previous
import functools

import jax
import jax.numpy as jnp
from jax import lax
from jax.experimental import pallas as pl
from jax.experimental.pallas import tpu as pltpu
from jax.experimental.pallas import tpu_sc as plsc

N = 1048576
_INFO = plsc.get_sparse_core_info()
NC = _INFO.num_cores
NS = _INFO.num_subcores
NW = NC * NS
CH = N // NW
NB = 1024
NR = NW * NB
NV = CH // 16
KMAX = (1 << 30) - 1
LN2 = 0.6931471805599453

_MESH = plsc.VectorSubcoreMesh(core_axis_name="c", subcore_axis_name="s")
_CPARAMS = pltpu.CompilerParams(needs_layout_passes=False)

_GATHER_DNUMS = lax.GatherDimensionNumbers(
    offset_dims=(), collapsed_slice_dims=(0,), start_index_map=(0,)
)


def _lane_bcast_last(x):
    idx = jnp.full((16, 1), 15, jnp.int32)
    return lax.gather(
        x, idx, _GATHER_DNUMS, slice_sizes=(1,),
        mode=lax.GatherScatterMode.PROMISE_IN_BOUNDS,
    )


def _worker_id():
    return lax.axis_index("s") * NC + lax.axis_index("c")


def _zero_i32(ref, n):
    z = jnp.zeros((16,), jnp.int32)

    def body(k, c):
        ref[pl.ds(k * 16, 16)] = z
        return c

    lax.fori_loop(0, n // 16, body, 0)


@functools.partial(
    pl.kernel,
    mesh=_MESH,
    compiler_params=_CPARAMS,
    out_type=(
        jax.ShapeDtypeStruct((N,), jnp.float32),
        jax.ShapeDtypeStruct((N,), jnp.float32),
        jax.ShapeDtypeStruct((NW, 4, 16), jnp.float32),
    ),
    scratch_types=[
        pltpu.VMEM((CH,), jnp.float32),
        pltpu.VMEM((CH,), jnp.float32),
        pltpu.VMEM((CH,), jnp.int32),
        pltpu.VMEM((4, 16), jnp.float32),
        pltpu.SemaphoreType.DMA,
    ],
)
def _k_pre(t_hbm, r_hbm, e_hbm, key_hbm, sw_hbm, part_hbm,
           t_v, r_v, e_v, part_v, sem):
    wid = _worker_id()
    base = wid * CH
    pltpu.sync_copy(t_hbm.at[pl.ds(base, CH)], t_v)
    pltpu.sync_copy(r_hbm.at[pl.ds(base, CH)], r_v)
    pltpu.sync_copy(e_hbm.at[pl.ds(base, CH)], e_v)

    def body(k, accs):
        ae, aer = accs
        sl = pl.ds(k * 16, 16)
        e16 = e_v[sl]
        r16 = r_v[sl]
        t16 = t_v[sl]
        ef = e16.astype(jnp.float32)
        ww = jnp.exp(r16)
        r_v[sl] = ww * (1.0 - 2.0 * ef)
        t_v[sl] = plsc.bitcast(KMAX - plsc.bitcast(t16, jnp.int32),
                               jnp.float32)
        return (ae + ef, aer + ef * r16)

    z = jnp.zeros((16,), jnp.float32)
    ae, aer = lax.fori_loop(0, NV, body, (z, z))
    part_v[0, :] = z
    part_v[1, :] = ae
    part_v[2, :] = aer
    part_v[3, :] = z
    pltpu.sync_copy(r_v, sw_hbm.at[pl.ds(base, CH)])
    pltpu.sync_copy(t_v, key_hbm.at[pl.ds(base, CH)])
    pltpu.sync_copy(part_v, part_hbm.at[wid])


def _make_group(shift):
    @functools.partial(
        pl.kernel,
        mesh=_MESH,
        compiler_params=_CPARAMS,
        out_type=(
            jax.ShapeDtypeStruct((N,), jnp.float32),
            jax.ShapeDtypeStruct((N,), jnp.float32),
            jax.ShapeDtypeStruct((NW, NB), jnp.int32),
        ),
        scratch_types=[
            pltpu.VMEM((CH,), jnp.float32),
            pltpu.VMEM((CH,), jnp.float32),
            pltpu.VMEM((CH,), jnp.float32),
            pltpu.VMEM((16 * NB,), jnp.int32),
            pltpu.VMEM((NB,), jnp.int32),
            pltpu.VMEM((NB,), jnp.int32),
            pltpu.SemaphoreType.DMA,
        ],
    )
    def _k_group(key_hbm, sw_hbm, keyo_hbm, swo_hbm, hist_hbm,
                 k_v, s_v, g_v, sub_v, m_v, offs_v, sem):
        wid = _worker_id()
        base = wid * CH
        pltpu.sync_copy(key_hbm.at[pl.ds(base, CH)], k_v)
        pltpu.sync_copy(sw_hbm.at[pl.ds(base, CH)], s_v)
        _zero_i32(sub_v, 16 * NB)
        lanes = lax.iota(jnp.int32, 16)
        ones = jnp.ones((16,), jnp.int32)

        def hist_body(k, c):
            kb = plsc.bitcast(k_v[pl.ds(k * 16, 16)], jnp.int32)
            d16 = (kb >> shift) & (NB - 1)
            plsc.addupdate_scatter(sub_v, [lanes * NB + d16], ones)
            return c

        lax.fori_loop(0, NV, hist_body, 0)

        def merge_body(c, carry):
            acc = jnp.zeros((16,), jnp.int32)
            for lane in range(16):
                acc = acc + sub_v[pl.ds(lane * NB + c * 16, 16)]
            m_v[pl.ds(c * 16, 16)] = acc
            return carry

        lax.fori_loop(0, NB // 16, merge_body, 0)

        def prefix_into_offs():
            def pfx(c, carry):
                sl = pl.ds(c * 16, 16)
                t16 = m_v[sl]
                pre = jnp.cumsum(t16) + carry
                offs_v[sl] = pre - t16
                return _lane_bcast_last(pre)

            lax.fori_loop(0, NB // 16, pfx, jnp.zeros((16,), jnp.int32))

        def scat_round(src_ref):
            def body(k, c):
                sl = pl.ds(k * 16, 16)
                kb = plsc.bitcast(k_v[sl], jnp.int32)
                d16 = (kb >> shift) & (NB - 1)
                sc, mlast = plsc.scan_count(d16)
                pos = plsc.load_gather(offs_v, [d16]) + sc - 1
                plsc.addupdate_scatter(offs_v, [d16], sc, mask=mlast)
                plsc.store_scatter(g_v, [pos], src_ref[sl])
                return c

            lax.fori_loop(0, NV, body, 0)

        prefix_into_offs()
        scat_round(k_v)
        pltpu.sync_copy(g_v, keyo_hbm.at[pl.ds(base, CH)])
        prefix_into_offs()
        scat_round(s_v)
        pltpu.sync_copy(g_v, swo_hbm.at[pl.ds(base, CH)])
        pltpu.sync_copy(m_v, hist_hbm.at[wid])

    return _k_group


def _make_build(shift, last):
    outs = [jax.ShapeDtypeStruct((N,), jnp.float32)]
    if last:
        outs.append(jax.ShapeDtypeStruct((NW, 4, 16), jnp.float32))
    else:
        outs.append(jax.ShapeDtypeStruct((N,), jnp.float32))

    @functools.partial(
        pl.kernel,
        mesh=_MESH,
        compiler_params=_CPARAMS,
        out_type=tuple(outs),
        scratch_types=[
            pltpu.VMEM((NR,), jnp.int32),
            pltpu.VMEM((NR,), jnp.float32),
            pltpu.VMEM((CH,), jnp.int32),
            pltpu.VMEM((4, 16), jnp.float32),
            pltpu.SemaphoreType.DMA,
        ],
    )
    def _k_build(keyg_hbm, swg_hbm, hist_hbm, swo_hbm, aux_hbm,
                 lp_v, vd_v, ri_v, part_v, sem):
        wid = _worker_id()
        base = wid * CH
        for w in range(NW):
            pltpu.sync_copy(hist_hbm.at[w], lp_v.at[pl.ds(w * NB, NB)])

        for w in range(NW):
            def pfx(c, carry, _w=w):
                sl = pl.ds(_w * NB + c * 16, 16)
                t16 = lp_v[sl]
                pre = jnp.cumsum(t16) + carry
                lp_v[sl] = pre - t16
                return _lane_bcast_last(pre)

            lax.fori_loop(0, NB // 16, pfx, jnp.zeros((16,), jnp.int32))

        _zero_i32(ri_v, CH)
        lanes = lax.iota(jnp.int32, 16)
        zi = jnp.zeros((16,), jnp.int32)
        base_vec = jnp.full((16,), base, jnp.int32)

        def run_body(q, st):
            gcarry, cover = st
            rho = q * 16 + lanes
            d16 = rho >> 5
            w16 = rho & (NW - 1)
            a16 = w16 * NB + d16
            lpv = plsc.load_gather(lp_v, [a16])
            is_last_d = d16 == (NB - 1)
            a2 = jnp.where(is_last_d, a16, a16 + 1)
            nxt = plsc.load_gather(lp_v, [a2])
            len16 = jnp.where(is_last_d, CH - lpv, nxt - lpv)
            pre = jnp.cumsum(len16) + gcarry
            g16 = pre - len16
            vd_v[pl.ds(q * 16, 16)] = plsc.bitcast(
                w16 * CH + lpv - g16, jnp.float32)
            real = len16 > zi
            inb = real & (g16 >= base_vec) & (g16 < base_vec + CH)
            plsc.store_scatter(ri_v, [g16 - base_vec], rho + 1, mask=inb)
            covc = real & (g16 <= base_vec)
            cover = jnp.maximum(cover, jnp.where(covc, rho + 1, zi))
            return (_lane_bcast_last(pre), cover)

        _, cover = lax.fori_loop(0, NR // 16, run_body, (zi, zi))
        fcarry0 = _lane_bcast_last(plsc.cummax(cover))

        def fill_body(k, fcarry):
            sl = pl.ds(k * 16, 16)
            filled = jnp.maximum(plsc.cummax(ri_v[sl]), fcarry)
            v16 = plsc.bitcast(
                plsc.load_gather(vd_v, [filled - 1]), jnp.int32)
            ri_v[sl] = v16 + (base + k * 16 + lanes)
            return _lane_bcast_last(filled)

        lax.fori_loop(0, NV, fill_body, fcarry0)

        LAG = 8

        def gather_to(src_hbm, dst_v):
            def body(j, c):
                sl = pl.ds(j * 128, 128)
                pltpu.async_copy(src_hbm.at[ri_v.at[sl]], dst_v.at[sl], sem)

                @pl.when(j >= LAG)
                def _():
                    sl2 = pl.ds((j - LAG) * 128, 128)
                    pltpu.make_async_copy(
                        src_hbm.at[ri_v.at[sl2]], dst_v.at[sl2], sem).wait()

                return c

            lax.fori_loop(0, CH // 128, body, 0)
            for u in range(LAG):
                sl2 = pl.ds((CH // 128 - LAG + u) * 128, 128)
                pltpu.make_async_copy(
                    src_hbm.at[ri_v.at[sl2]], dst_v.at[sl2], sem).wait()

        gather_to(swg_hbm, vd_v)
        pltpu.sync_copy(vd_v, swo_hbm.at[pl.ds(base, CH)])
        if last:
            def sum_body(k, acc):
                b = plsc.bitcast(vd_v[pl.ds(k * 16, 16)], jnp.int32)
                return acc + plsc.bitcast(b & 0x7FFFFFFF, jnp.float32)

            zf = jnp.zeros((16,), jnp.float32)
            acc = lax.fori_loop(0, NV, sum_body, zf)
            part_v[0, :] = acc
            part_v[1, :] = zf
            part_v[2, :] = zf
            part_v[3, :] = zf
            pltpu.sync_copy(part_v, aux_hbm.at[wid])
        else:
            gather_to(keyg_hbm, vd_v)
            pltpu.sync_copy(vd_v, aux_hbm.at[pl.ds(base, CH)])

    return _k_build


@functools.partial(
    pl.kernel,
    mesh=_MESH,
    compiler_params=_CPARAMS,
    out_type=jax.ShapeDtypeStruct((NW, 16), jnp.float32),
    scratch_types=[
        pltpu.VMEM((CH,), jnp.float32),
        pltpu.VMEM((NW, 4, 16), jnp.float32),
        pltpu.VMEM((16,), jnp.float32),
        pltpu.SemaphoreType.DMA,
    ],
)
def _k_log(w_hbm, part_hbm, out_hbm, w_v, part_v, out_v, sem):
    wid = _worker_id()
    base = wid * CH
    pltpu.sync_copy(w_hbm.at[pl.ds(base, CH)], w_v)
    pltpu.sync_copy(part_hbm, part_v)

    wid_vec = jnp.full((16,), wid, jnp.int32)
    pacc = jnp.zeros((16,), jnp.float32)
    for v in range(NW):
        sel = jnp.full((16,), v, jnp.int32) < wid_vec
        pacc = pacc + jnp.where(sel, part_v[v, 0, :], 0.0)
    carry0 = _lane_bcast_last(jnp.cumsum(pacc))

    def body(k, st):
        cvec, acc = st
        swv = w_v[pl.ds(k * 16, 16)]
        b = plsc.bitcast(swv, jnp.int32)
        ww = plsc.bitcast(b & 0x7FFFFFFF, jnp.float32)
        ef = lax.shift_right_logical(b, 31).astype(jnp.float32)
        pre = jnp.cumsum(ww) + cvec
        cnew = _lane_bcast_last(pre)
        pb = plsc.bitcast(pre, jnp.int32)
        ex = lax.shift_right_logical(pb, 23) - 127
        m = plsc.bitcast((pb & 0x7FFFFF) | 0x3F800000, jnp.float32)
        big = m >= 1.5
        m = jnp.where(big, m * 0.5, m)
        exf = (ex + big.astype(jnp.int32)).astype(jnp.float32)
        s = (m - 1.0) / (m + 1.0)
        s2 = s * s
        lnm = 2.0 * s * (1.0 + s2 * (1.0 / 3.0 + s2 * 0.2))
        lnx = exf * LN2 + lnm
        return (cnew, acc + ef * lnx)

    _, acc = lax.fori_loop(
        0, NV, body, (carry0, jnp.zeros((16,), jnp.float32))
    )
    out_v[...] = acc
    pltpu.sync_copy(out_v, out_hbm.at[wid])


_group1 = _make_group(0)
_build1 = _make_build(0, False)
_group2 = _make_group(10)
_build2 = _make_build(10, False)
_group3 = _make_group(20)
_build3 = _make_build(20, True)


def kernel(risk_scores, events, survival_times):
    key0, sw0, parts = _k_pre(survival_times, risk_scores, events)
    kg1, sg1, h1 = _group1(key0, sw0)
    sw1, key1 = _build1(kg1, sg1, h1)
    kg2, sg2, h2 = _group2(key1, sw1)
    sw2, key2 = _build2(kg2, sg2, h2)
    kg3, sg3, h3 = _group3(key2, sw2)
    sw3, parts2 = _build3(kg3, sg3, h3)
    accs = _k_log(sw3, parts2)
    sum_e = parts[:, 1, :].sum()
    sum_er = parts[:, 2, :].sum()
    sum_elogc = accs.sum()
    return (sum_elogc - sum_er) / sum_e


# speedup vs baseline: 4.9322x; 1.0371x over previous
import functools

import jax
import jax.numpy as jnp
from jax import lax
from jax.experimental import pallas as pl
from jax.experimental.pallas import tpu as pltpu
from jax.experimental.pallas import tpu_sc as plsc

N = 1048576
_INFO = plsc.get_sparse_core_info()
NC = _INFO.num_cores
NS = _INFO.num_subcores
NW = NC * NS
CH = N // NW
NB = 1024
NR = NW * NB
NV = CH // 16
KMAX = (1 << 30) - 1
LN2 = 0.6931471805599453

_MESH = plsc.VectorSubcoreMesh(core_axis_name="c", subcore_axis_name="s")
_CPARAMS = pltpu.CompilerParams(needs_layout_passes=False)

_GATHER_DNUMS = lax.GatherDimensionNumbers(
    offset_dims=(), collapsed_slice_dims=(0,), start_index_map=(0,)
)


def _lane_bcast_last(x):
    idx = jnp.full((16, 1), 15, jnp.int32)
    return lax.gather(
        x, idx, _GATHER_DNUMS, slice_sizes=(1,),
        mode=lax.GatherScatterMode.PROMISE_IN_BOUNDS,
    )


def _worker_id():
    return lax.axis_index("s") * NC + lax.axis_index("c")


def _zero_i32(ref, n):
    z = jnp.zeros((16,), jnp.int32)

    def body(k, c):
        ref[pl.ds(k * 16, 16)] = z
        return c

    lax.fori_loop(0, n // 16, body, 0)


@functools.partial(
    pl.kernel,
    mesh=_MESH,
    compiler_params=_CPARAMS,
    out_type=(
        jax.ShapeDtypeStruct((N,), jnp.float32),
        jax.ShapeDtypeStruct((N,), jnp.float32),
        jax.ShapeDtypeStruct((NW, 4, 16), jnp.float32),
    ),
    scratch_types=[
        pltpu.VMEM((CH,), jnp.float32),
        pltpu.VMEM((CH,), jnp.float32),
        pltpu.VMEM((CH,), jnp.int32),
        pltpu.VMEM((4, 16), jnp.float32),
        pltpu.SemaphoreType.DMA,
    ],
)
def _k_pre(t_hbm, r_hbm, e_hbm, key_hbm, sw_hbm, part_hbm,
           t_v, r_v, e_v, part_v, sem):
    wid = _worker_id()
    base = wid * CH
    pltpu.sync_copy(t_hbm.at[pl.ds(base, CH)], t_v)
    pltpu.sync_copy(r_hbm.at[pl.ds(base, CH)], r_v)
    pltpu.sync_copy(e_hbm.at[pl.ds(base, CH)], e_v)

    def body(k, accs):
        ae, aer = accs
        sl = pl.ds(k * 16, 16)
        e16 = e_v[sl]
        r16 = r_v[sl]
        t16 = t_v[sl]
        ef = e16.astype(jnp.float32)
        ww = jnp.exp(r16)
        r_v[sl] = ww * (1.0 - 2.0 * ef)
        t_v[sl] = plsc.bitcast(KMAX - plsc.bitcast(t16, jnp.int32),
                               jnp.float32)
        return (ae + ef, aer + ef * r16)

    z = jnp.zeros((16,), jnp.float32)
    ae, aer = lax.fori_loop(0, NV, body, (z, z))
    part_v[0, :] = z
    part_v[1, :] = ae
    part_v[2, :] = aer
    part_v[3, :] = z
    pltpu.sync_copy(r_v, sw_hbm.at[pl.ds(base, CH)])
    pltpu.sync_copy(t_v, key_hbm.at[pl.ds(base, CH)])
    pltpu.sync_copy(part_v, part_hbm.at[wid])


def _make_group(shift):
    @functools.partial(
        pl.kernel,
        mesh=_MESH,
        compiler_params=_CPARAMS,
        out_type=(
            jax.ShapeDtypeStruct((N,), jnp.float32),
            jax.ShapeDtypeStruct((N,), jnp.float32),
            jax.ShapeDtypeStruct((NW, NB), jnp.int32),
        ),
        scratch_types=[
            pltpu.VMEM((CH,), jnp.float32),
            pltpu.VMEM((CH,), jnp.float32),
            pltpu.VMEM((CH,), jnp.float32),
            pltpu.VMEM((16 * NB,), jnp.int32),
            pltpu.VMEM((NB,), jnp.int32),
            pltpu.VMEM((NB,), jnp.int32),
            pltpu.SemaphoreType.DMA,
        ],
    )
    def _k_group(key_hbm, sw_hbm, keyo_hbm, swo_hbm, hist_hbm,
                 k_v, s_v, g_v, sub_v, m_v, offs_v, sem):
        wid = _worker_id()
        base = wid * CH
        pltpu.sync_copy(key_hbm.at[pl.ds(base, CH)], k_v)
        pltpu.sync_copy(sw_hbm.at[pl.ds(base, CH)], s_v)
        _zero_i32(sub_v, 16 * NB)
        lanes = lax.iota(jnp.int32, 16)
        ones = jnp.ones((16,), jnp.int32)

        def hist_body(k, c):
            kb = plsc.bitcast(k_v[pl.ds(k * 16, 16)], jnp.int32)
            d16 = (kb >> shift) & (NB - 1)
            plsc.addupdate_scatter(sub_v, [lanes * NB + d16], ones)
            return c

        lax.fori_loop(0, NV, hist_body, 0)

        def merge_body(c, carry):
            acc = jnp.zeros((16,), jnp.int32)
            for lane in range(16):
                acc = acc + sub_v[pl.ds(lane * NB + c * 16, 16)]
            m_v[pl.ds(c * 16, 16)] = acc
            return carry

        lax.fori_loop(0, NB // 16, merge_body, 0)

        def prefix_into_offs():
            def pfx(c, carry):
                sl = pl.ds(c * 16, 16)
                t16 = m_v[sl]
                pre = jnp.cumsum(t16) + carry
                offs_v[sl] = pre - t16
                return _lane_bcast_last(pre)

            lax.fori_loop(0, NB // 16, pfx, jnp.zeros((16,), jnp.int32))

        def scat_round(src_ref):
            def body(k, c):
                sl = pl.ds(k * 16, 16)
                kb = plsc.bitcast(k_v[sl], jnp.int32)
                d16 = (kb >> shift) & (NB - 1)
                sc, mlast = plsc.scan_count(d16)
                pos = plsc.load_gather(offs_v, [d16]) + sc - 1
                plsc.addupdate_scatter(offs_v, [d16], sc, mask=mlast)
                plsc.store_scatter(g_v, [pos], src_ref[sl])
                return c

            lax.fori_loop(0, NV, body, 0)

        prefix_into_offs()
        scat_round(k_v)
        pltpu.sync_copy(g_v, keyo_hbm.at[pl.ds(base, CH)])
        prefix_into_offs()
        scat_round(s_v)
        pltpu.sync_copy(g_v, swo_hbm.at[pl.ds(base, CH)])
        pltpu.sync_copy(m_v, hist_hbm.at[wid])

    return _k_group


def _make_build(shift, last):
    outs = [jax.ShapeDtypeStruct((N,), jnp.float32)]
    if last:
        outs.append(jax.ShapeDtypeStruct((NW, 4, 16), jnp.float32))
    else:
        outs.append(jax.ShapeDtypeStruct((N,), jnp.float32))

    @functools.partial(
        pl.kernel,
        mesh=_MESH,
        compiler_params=_CPARAMS,
        out_type=tuple(outs),
        scratch_types=[
            pltpu.VMEM((NR,), jnp.int32),
            pltpu.VMEM((NR,), jnp.float32),
            pltpu.VMEM((CH,), jnp.int32),
            pltpu.VMEM((4, 16), jnp.float32),
            pltpu.SemaphoreType.DMA,
        ],
    )
    def _k_build(keyg_hbm, swg_hbm, hist_hbm, swo_hbm, aux_hbm,
                 lp_v, vd_v, ri_v, part_v, sem):
        wid = _worker_id()
        base = wid * CH
        for w in range(NW):
            pltpu.sync_copy(hist_hbm.at[w], lp_v.at[pl.ds(w * NB, NB)])

        for w in range(NW):
            def pfx(c, carry, _w=w):
                sl = pl.ds(_w * NB + c * 16, 16)
                t16 = lp_v[sl]
                pre = jnp.cumsum(t16) + carry
                lp_v[sl] = pre - t16
                return _lane_bcast_last(pre)

            lax.fori_loop(0, NB // 16, pfx, jnp.zeros((16,), jnp.int32))

        _zero_i32(ri_v, CH)
        lanes = lax.iota(jnp.int32, 16)
        zi = jnp.zeros((16,), jnp.int32)
        base_vec = jnp.full((16,), base, jnp.int32)

        def run_body(q, st):
            gcarry, cover = st
            rho = q * 16 + lanes
            d16 = rho >> 5
            w16 = rho & (NW - 1)
            a16 = w16 * NB + d16
            lpv = plsc.load_gather(lp_v, [a16])
            is_last_d = d16 == (NB - 1)
            a2 = jnp.where(is_last_d, a16, a16 + 1)
            nxt = plsc.load_gather(lp_v, [a2])
            len16 = jnp.where(is_last_d, CH - lpv, nxt - lpv)
            pre = jnp.cumsum(len16) + gcarry
            g16 = pre - len16
            vd_v[pl.ds(q * 16, 16)] = plsc.bitcast(
                w16 * CH + lpv - g16, jnp.float32)
            real = len16 > zi
            inb = real & (g16 >= base_vec) & (g16 < base_vec + CH)
            plsc.store_scatter(ri_v, [g16 - base_vec], rho + 1, mask=inb)
            covc = real & (g16 <= base_vec)
            cover = jnp.maximum(cover, jnp.where(covc, rho + 1, zi))
            return (_lane_bcast_last(pre), cover)

        _, cover = lax.fori_loop(0, NR // 16, run_body, (zi, zi))
        fcarry0 = _lane_bcast_last(plsc.cummax(cover))

        def fill_body(k, fcarry):
            sl = pl.ds(k * 16, 16)
            filled = jnp.maximum(plsc.cummax(ri_v[sl]), fcarry)
            v16 = plsc.bitcast(
                plsc.load_gather(vd_v, [filled - 1]), jnp.int32)
            ri_v[sl] = v16 + (base + k * 16 + lanes)
            return _lane_bcast_last(filled)

        lax.fori_loop(0, NV, fill_body, fcarry0)

        LAG = 8
        GB = 512

        def gather_to(src_hbm, dst_v):
            def body(j, c):
                sl = pl.ds(j * GB, GB)
                pltpu.async_copy(src_hbm.at[ri_v.at[sl]], dst_v.at[sl], sem)

                @pl.when(j >= LAG)
                def _():
                    sl2 = pl.ds((j - LAG) * GB, GB)
                    pltpu.make_async_copy(
                        src_hbm.at[ri_v.at[sl2]], dst_v.at[sl2], sem).wait()

                return c

            lax.fori_loop(0, CH // GB, body, 0)
            for u in range(LAG):
                sl2 = pl.ds((CH // GB - LAG + u) * GB, GB)
                pltpu.make_async_copy(
                    src_hbm.at[ri_v.at[sl2]], dst_v.at[sl2], sem).wait()

        gather_to(swg_hbm, vd_v)
        pltpu.sync_copy(vd_v, swo_hbm.at[pl.ds(base, CH)])
        if last:
            def sum_body(k, acc):
                b = plsc.bitcast(vd_v[pl.ds(k * 16, 16)], jnp.int32)
                return acc + plsc.bitcast(b & 0x7FFFFFFF, jnp.float32)

            zf = jnp.zeros((16,), jnp.float32)
            acc = lax.fori_loop(0, NV, sum_body, zf)
            part_v[0, :] = acc
            part_v[1, :] = zf
            part_v[2, :] = zf
            part_v[3, :] = zf
            pltpu.sync_copy(part_v, aux_hbm.at[wid])
        else:
            gather_to(keyg_hbm, vd_v)
            pltpu.sync_copy(vd_v, aux_hbm.at[pl.ds(base, CH)])

    return _k_build


@functools.partial(
    pl.kernel,
    mesh=_MESH,
    compiler_params=_CPARAMS,
    out_type=jax.ShapeDtypeStruct((NW, 16), jnp.float32),
    scratch_types=[
        pltpu.VMEM((CH,), jnp.float32),
        pltpu.VMEM((NW, 4, 16), jnp.float32),
        pltpu.VMEM((16,), jnp.float32),
        pltpu.SemaphoreType.DMA,
    ],
)
def _k_log(w_hbm, part_hbm, out_hbm, w_v, part_v, out_v, sem):
    wid = _worker_id()
    base = wid * CH
    pltpu.sync_copy(w_hbm.at[pl.ds(base, CH)], w_v)
    pltpu.sync_copy(part_hbm, part_v)

    wid_vec = jnp.full((16,), wid, jnp.int32)
    pacc = jnp.zeros((16,), jnp.float32)
    for v in range(NW):
        sel = jnp.full((16,), v, jnp.int32) < wid_vec
        pacc = pacc + jnp.where(sel, part_v[v, 0, :], 0.0)
    carry0 = _lane_bcast_last(jnp.cumsum(pacc))

    def body(k, st):
        cvec, acc = st
        swv = w_v[pl.ds(k * 16, 16)]
        b = plsc.bitcast(swv, jnp.int32)
        ww = plsc.bitcast(b & 0x7FFFFFFF, jnp.float32)
        ef = lax.shift_right_logical(b, 31).astype(jnp.float32)
        pre = jnp.cumsum(ww) + cvec
        cnew = _lane_bcast_last(pre)
        pb = plsc.bitcast(pre, jnp.int32)
        ex = lax.shift_right_logical(pb, 23) - 127
        m = plsc.bitcast((pb & 0x7FFFFF) | 0x3F800000, jnp.float32)
        big = m >= 1.5
        m = jnp.where(big, m * 0.5, m)
        exf = (ex + big.astype(jnp.int32)).astype(jnp.float32)
        s = (m - 1.0) / (m + 1.0)
        s2 = s * s
        lnm = 2.0 * s * (1.0 + s2 * (1.0 / 3.0 + s2 * 0.2))
        lnx = exf * LN2 + lnm
        return (cnew, acc + ef * lnx)

    _, acc = lax.fori_loop(
        0, NV, body, (carry0, jnp.zeros((16,), jnp.float32))
    )
    out_v[...] = acc
    pltpu.sync_copy(out_v, out_hbm.at[wid])


_group1 = _make_group(0)
_build1 = _make_build(0, False)
_group2 = _make_group(10)
_build2 = _make_build(10, False)
_group3 = _make_group(20)
_build3 = _make_build(20, True)


def kernel(risk_scores, events, survival_times):
    key0, sw0, parts = _k_pre(survival_times, risk_scores, events)
    kg1, sg1, h1 = _group1(key0, sw0)
    sw1, key1 = _build1(kg1, sg1, h1)
    kg2, sg2, h2 = _group2(key1, sw1)
    sw2, key2 = _build2(kg2, sg2, h2)
    kg3, sg3, h3 = _group3(key2, sw2)
    sw3, parts2 = _build3(kg3, sg3, h3)
    accs = _k_log(sw3, parts2)
    sum_e = parts[:, 1, :].sum()
    sum_er = parts[:, 2, :].sum()
    sum_elogc = accs.sum()
    return (sum_elogc - sum_er) / sum_e


# speedup vs baseline: 4.9511x; 1.0038x over previous
import functools

import jax
import jax.numpy as jnp
from jax import lax
from jax.experimental import pallas as pl
from jax.experimental.pallas import tpu as pltpu
from jax.experimental.pallas import tpu_sc as plsc

N = 1048576
_INFO = plsc.get_sparse_core_info()
NC = _INFO.num_cores
NS = _INFO.num_subcores
NW = NC * NS
CH = N // NW
NB = 1024
NR = NW * NB
NV = CH // 16
KMAX = (1 << 30) - 1
LN2 = 0.6931471805599453

_MESH = plsc.VectorSubcoreMesh(core_axis_name="c", subcore_axis_name="s")
_CPARAMS = pltpu.CompilerParams(needs_layout_passes=False)

_GATHER_DNUMS = lax.GatherDimensionNumbers(
    offset_dims=(), collapsed_slice_dims=(0,), start_index_map=(0,)
)


def _lane_bcast_last(x):
    idx = jnp.full((16, 1), 15, jnp.int32)
    return lax.gather(
        x, idx, _GATHER_DNUMS, slice_sizes=(1,),
        mode=lax.GatherScatterMode.PROMISE_IN_BOUNDS,
    )


def _worker_id():
    return lax.axis_index("s") * NC + lax.axis_index("c")


def _zero_i32(ref, n):
    z = jnp.zeros((16,), jnp.int32)

    def body(k, c):
        ref[pl.ds(k * 16, 16)] = z
        return c

    lax.fori_loop(0, n // 16, body, 0)


@functools.partial(
    pl.kernel,
    mesh=_MESH,
    compiler_params=_CPARAMS,
    out_type=(
        jax.ShapeDtypeStruct((N,), jnp.float32),
        jax.ShapeDtypeStruct((N,), jnp.float32),
        jax.ShapeDtypeStruct((NW, 4, 16), jnp.float32),
    ),
    scratch_types=[
        pltpu.VMEM((CH,), jnp.float32),
        pltpu.VMEM((CH,), jnp.float32),
        pltpu.VMEM((CH,), jnp.int32),
        pltpu.VMEM((4, 16), jnp.float32),
        pltpu.SemaphoreType.DMA,
    ],
)
def _k_pre(t_hbm, r_hbm, e_hbm, key_hbm, sw_hbm, part_hbm,
           t_v, r_v, e_v, part_v, sem):
    wid = _worker_id()
    base = wid * CH
    pltpu.sync_copy(t_hbm.at[pl.ds(base, CH)], t_v)
    pltpu.sync_copy(r_hbm.at[pl.ds(base, CH)], r_v)
    pltpu.sync_copy(e_hbm.at[pl.ds(base, CH)], e_v)

    def body(k, accs):
        ae, aer = accs
        sl = pl.ds(k * 16, 16)
        e16 = e_v[sl]
        r16 = r_v[sl]
        t16 = t_v[sl]
        ef = e16.astype(jnp.float32)
        ww = jnp.exp(r16)
        r_v[sl] = ww * (1.0 - 2.0 * ef)
        t_v[sl] = plsc.bitcast(KMAX - plsc.bitcast(t16, jnp.int32),
                               jnp.float32)
        return (ae + ef, aer + ef * r16)

    z = jnp.zeros((16,), jnp.float32)
    ae, aer = lax.fori_loop(0, NV, body, (z, z))
    part_v[0, :] = z
    part_v[1, :] = ae
    part_v[2, :] = aer
    part_v[3, :] = z
    pltpu.sync_copy(r_v, sw_hbm.at[pl.ds(base, CH)])
    pltpu.sync_copy(t_v, key_hbm.at[pl.ds(base, CH)])
    pltpu.sync_copy(part_v, part_hbm.at[wid])


def _make_group(shift):
    @functools.partial(
        pl.kernel,
        mesh=_MESH,
        compiler_params=_CPARAMS,
        out_type=(
            jax.ShapeDtypeStruct((N,), jnp.float32),
            jax.ShapeDtypeStruct((N,), jnp.float32),
            jax.ShapeDtypeStruct((NW, NB), jnp.int32),
        ),
        scratch_types=[
            pltpu.VMEM((CH,), jnp.float32),
            pltpu.VMEM((CH,), jnp.float32),
            pltpu.VMEM((CH,), jnp.float32),
            pltpu.VMEM((16 * NB,), jnp.int32),
            pltpu.VMEM((NB,), jnp.int32),
            pltpu.VMEM((NB,), jnp.int32),
            pltpu.SemaphoreType.DMA,
        ],
    )
    def _k_group(key_hbm, sw_hbm, keyo_hbm, swo_hbm, hist_hbm,
                 k_v, s_v, g_v, sub_v, m_v, offs_v, sem):
        wid = _worker_id()
        base = wid * CH
        pltpu.sync_copy(key_hbm.at[pl.ds(base, CH)], k_v)
        pltpu.sync_copy(sw_hbm.at[pl.ds(base, CH)], s_v)
        _zero_i32(sub_v, 16 * NB)
        lanes = lax.iota(jnp.int32, 16)
        ones = jnp.ones((16,), jnp.int32)

        def hist_body(k, c):
            kb = plsc.bitcast(k_v[pl.ds(k * 16, 16)], jnp.int32)
            d16 = (kb >> shift) & (NB - 1)
            plsc.addupdate_scatter(sub_v, [lanes * NB + d16], ones)
            return c

        lax.fori_loop(0, NV, hist_body, 0)

        def merge_body(c, carry):
            acc = jnp.zeros((16,), jnp.int32)
            for lane in range(16):
                acc = acc + sub_v[pl.ds(lane * NB + c * 16, 16)]
            m_v[pl.ds(c * 16, 16)] = acc
            return carry

        lax.fori_loop(0, NB // 16, merge_body, 0)

        def prefix_into_offs():
            def pfx(c, carry):
                sl = pl.ds(c * 16, 16)
                t16 = m_v[sl]
                pre = jnp.cumsum(t16) + carry
                offs_v[sl] = pre - t16
                return _lane_bcast_last(pre)

            lax.fori_loop(0, NB // 16, pfx, jnp.zeros((16,), jnp.int32))

        def scat_round(src_ref):
            def body(k, c):
                sl = pl.ds(k * 16, 16)
                kb = plsc.bitcast(k_v[sl], jnp.int32)
                d16 = (kb >> shift) & (NB - 1)
                sc, mlast = plsc.scan_count(d16)
                pos = plsc.load_gather(offs_v, [d16]) + sc - 1
                plsc.addupdate_scatter(offs_v, [d16], sc, mask=mlast)
                plsc.store_scatter(g_v, [pos], src_ref[sl])
                return c

            lax.fori_loop(0, NV, body, 0)

        prefix_into_offs()
        scat_round(k_v)
        pltpu.sync_copy(g_v, keyo_hbm.at[pl.ds(base, CH)])
        prefix_into_offs()
        scat_round(s_v)
        pltpu.sync_copy(g_v, swo_hbm.at[pl.ds(base, CH)])
        pltpu.sync_copy(m_v, hist_hbm.at[wid])

    return _k_group


def _make_build(shift, last):
    outs = [jax.ShapeDtypeStruct((N,), jnp.float32)]
    if last:
        outs.append(jax.ShapeDtypeStruct((NW, 4, 16), jnp.float32))
    else:
        outs.append(jax.ShapeDtypeStruct((N,), jnp.float32))

    @functools.partial(
        pl.kernel,
        mesh=_MESH,
        compiler_params=_CPARAMS,
        out_type=tuple(outs),
        scratch_types=[
            pltpu.VMEM((NR,), jnp.int32),
            pltpu.VMEM((NR,), jnp.float32),
            pltpu.VMEM((CH,), jnp.int32),
            pltpu.VMEM((4, 16), jnp.float32),
            pltpu.SemaphoreType.DMA,
        ],
    )
    def _k_build(keyg_hbm, swg_hbm, hist_hbm, swo_hbm, aux_hbm,
                 lp_v, vd_v, ri_v, part_v, sem):
        wid = _worker_id()
        base = wid * CH
        for w in range(NW):
            pltpu.sync_copy(hist_hbm.at[w], lp_v.at[pl.ds(w * NB, NB)])

        for w in range(NW):
            def pfx(c, carry, _w=w):
                sl = pl.ds(_w * NB + c * 16, 16)
                t16 = lp_v[sl]
                pre = jnp.cumsum(t16) + carry
                lp_v[sl] = pre - t16
                return _lane_bcast_last(pre)

            lax.fori_loop(0, NB // 16, pfx, jnp.zeros((16,), jnp.int32))

        _zero_i32(ri_v, CH)
        lanes = lax.iota(jnp.int32, 16)
        zi = jnp.zeros((16,), jnp.int32)
        base_vec = jnp.full((16,), base, jnp.int32)

        def run_body(q, st):
            gcarry, cover = st
            rho = q * 16 + lanes
            d16 = rho >> 5
            w16 = rho & (NW - 1)
            a16 = w16 * NB + d16
            lpv = plsc.load_gather(lp_v, [a16])
            is_last_d = d16 == (NB - 1)
            a2 = jnp.where(is_last_d, a16, a16 + 1)
            nxt = plsc.load_gather(lp_v, [a2])
            len16 = jnp.where(is_last_d, CH - lpv, nxt - lpv)
            pre = jnp.cumsum(len16) + gcarry
            g16 = pre - len16
            vd_v[pl.ds(q * 16, 16)] = plsc.bitcast(
                w16 * CH + lpv - g16, jnp.float32)
            real = len16 > zi
            inb = real & (g16 >= base_vec) & (g16 < base_vec + CH)
            plsc.store_scatter(ri_v, [g16 - base_vec], rho + 1, mask=inb)
            covc = real & (g16 <= base_vec)
            cover = jnp.maximum(cover, jnp.where(covc, rho + 1, zi))
            return (_lane_bcast_last(pre), cover)

        _, cover = lax.fori_loop(0, NR // 16, run_body, (zi, zi))
        fcarry0 = _lane_bcast_last(plsc.cummax(cover))

        def fill_body(k, fcarry):
            sl = pl.ds(k * 16, 16)
            filled = jnp.maximum(plsc.cummax(ri_v[sl]), fcarry)
            v16 = plsc.bitcast(
                plsc.load_gather(vd_v, [filled - 1]), jnp.int32)
            ri_v[sl] = v16 + (base + k * 16 + lanes)
            return _lane_bcast_last(filled)

        lax.fori_loop(0, NV, fill_body, fcarry0)

        LAG = 4
        GB = 1024

        def gather_to(src_hbm, dst_v):
            def body(j, c):
                sl = pl.ds(j * GB, GB)
                pltpu.async_copy(src_hbm.at[ri_v.at[sl]], dst_v.at[sl], sem)

                @pl.when(j >= LAG)
                def _():
                    sl2 = pl.ds((j - LAG) * GB, GB)
                    pltpu.make_async_copy(
                        src_hbm.at[ri_v.at[sl2]], dst_v.at[sl2], sem).wait()

                return c

            lax.fori_loop(0, CH // GB, body, 0)
            for u in range(LAG):
                sl2 = pl.ds((CH // GB - LAG + u) * GB, GB)
                pltpu.make_async_copy(
                    src_hbm.at[ri_v.at[sl2]], dst_v.at[sl2], sem).wait()

        gather_to(swg_hbm, vd_v)
        pltpu.sync_copy(vd_v, swo_hbm.at[pl.ds(base, CH)])
        if last:
            def sum_body(k, acc):
                b = plsc.bitcast(vd_v[pl.ds(k * 16, 16)], jnp.int32)
                return acc + plsc.bitcast(b & 0x7FFFFFFF, jnp.float32)

            zf = jnp.zeros((16,), jnp.float32)
            acc = lax.fori_loop(0, NV, sum_body, zf)
            part_v[0, :] = acc
            part_v[1, :] = zf
            part_v[2, :] = zf
            part_v[3, :] = zf
            pltpu.sync_copy(part_v, aux_hbm.at[wid])
        else:
            gather_to(keyg_hbm, vd_v)
            pltpu.sync_copy(vd_v, aux_hbm.at[pl.ds(base, CH)])

    return _k_build


@functools.partial(
    pl.kernel,
    mesh=_MESH,
    compiler_params=_CPARAMS,
    out_type=jax.ShapeDtypeStruct((NW, 16), jnp.float32),
    scratch_types=[
        pltpu.VMEM((CH,), jnp.float32),
        pltpu.VMEM((NW, 4, 16), jnp.float32),
        pltpu.VMEM((16,), jnp.float32),
        pltpu.SemaphoreType.DMA,
    ],
)
def _k_log(w_hbm, part_hbm, out_hbm, w_v, part_v, out_v, sem):
    wid = _worker_id()
    base = wid * CH
    pltpu.sync_copy(w_hbm.at[pl.ds(base, CH)], w_v)
    pltpu.sync_copy(part_hbm, part_v)

    wid_vec = jnp.full((16,), wid, jnp.int32)
    pacc = jnp.zeros((16,), jnp.float32)
    for v in range(NW):
        sel = jnp.full((16,), v, jnp.int32) < wid_vec
        pacc = pacc + jnp.where(sel, part_v[v, 0, :], 0.0)
    carry0 = _lane_bcast_last(jnp.cumsum(pacc))

    def body(k, st):
        cvec, acc = st
        swv = w_v[pl.ds(k * 16, 16)]
        b = plsc.bitcast(swv, jnp.int32)
        ww = plsc.bitcast(b & 0x7FFFFFFF, jnp.float32)
        ef = lax.shift_right_logical(b, 31).astype(jnp.float32)
        pre = jnp.cumsum(ww) + cvec
        cnew = _lane_bcast_last(pre)
        pb = plsc.bitcast(pre, jnp.int32)
        ex = lax.shift_right_logical(pb, 23) - 127
        m = plsc.bitcast((pb & 0x7FFFFF) | 0x3F800000, jnp.float32)
        big = m >= 1.5
        m = jnp.where(big, m * 0.5, m)
        exf = (ex + big.astype(jnp.int32)).astype(jnp.float32)
        s = (m - 1.0) / (m + 1.0)
        s2 = s * s
        lnm = 2.0 * s * (1.0 + s2 * (1.0 / 3.0 + s2 * 0.2))
        lnx = exf * LN2 + lnm
        return (cnew, acc + ef * lnx)

    _, acc = lax.fori_loop(
        0, NV, body, (carry0, jnp.zeros((16,), jnp.float32))
    )
    out_v[...] = acc
    pltpu.sync_copy(out_v, out_hbm.at[wid])


_group1 = _make_group(0)
_build1 = _make_build(0, False)
_group2 = _make_group(10)
_build2 = _make_build(10, False)
_group3 = _make_group(20)
_build3 = _make_build(20, True)


def kernel(risk_scores, events, survival_times):
    key0, sw0, parts = _k_pre(survival_times, risk_scores, events)
    kg1, sg1, h1 = _group1(key0, sw0)
    sw1, key1 = _build1(kg1, sg1, h1)
    kg2, sg2, h2 = _group2(key1, sw1)
    sw2, key2 = _build2(kg2, sg2, h2)
    kg3, sg3, h3 = _group3(key2, sw2)
    sw3, parts2 = _build3(kg3, sg3, h3)
    accs = _k_log(sw3, parts2)
    sum_e = parts[:, 1, :].sum()
    sum_er = parts[:, 2, :].sum()
    sum_elogc = accs.sum()
    return (sum_elogc - sum_er) / sum_e
